# baseline (speedup 1.0000x reference)
import functools

import jax
import jax.numpy as jnp
from jax.experimental import pallas as pl
from jax.experimental.pallas import tpu as pltpu

_BATCH = 8192
_D = 2048
_N_CLUSTERS = 3
_ITERS = 20
_EPS = 1e-7
_BAND = 0.01
_BR = 512
_NB = _BATCH // _BR


def _onehot(dis):
    r1 = jnp.roll(dis, -1, axis=1)
    r2 = jnp.roll(dis, -2, axis=1)
    idx = jax.lax.broadcasted_iota(jnp.int32, (1, _N_CLUSTERS), 1)
    i1 = jnp.roll(idx, -1, axis=1)
    i2 = jnp.roll(idx, -2, axis=1)
    beat1 = (dis < r1) | ((dis == r1) & (idx < i1))
    beat2 = (dis < r2) | ((dis == r2) & (idx < i2))
    return (beat1 & beat2).astype(jnp.float32)


def _kmeans_step(zbf, i, state_ref, total_ref, count_ref):
    stT = state_ref[...]
    scores = jax.lax.dot_general(
        zbf, stT.astype(jnp.bfloat16), (((1,), (0,)), ((), ())),
        preferred_element_type=jnp.float32)
    ssq = jnp.sum(stT * stT, axis=0, keepdims=True)
    dis = ssq - 2.0 * scores
    oh = _onehot(dis)
    total_ref[...] += jax.lax.dot_general(
        zbf, oh.astype(jnp.bfloat16), (((0,), (0,)), ((), ())),
        preferred_element_type=jnp.float32)
    count_ref[...] += jnp.sum(oh, axis=0, keepdims=True)

    @pl.when(i == _NB - 1)
    def _update_centers():
        state_ref[...] = total_ref[...] / count_ref[...]
        total_ref[...] = jnp.zeros_like(total_ref)
        count_ref[...] = jnp.zeros_like(count_ref)


def _body(state0_ref, u_ref, w_ref, b_ref, z_ref,
          fz_ref, ld_ref, pen_ref,
          state_ref, total_ref, count_ref, res_ref):
    t = pl.program_id(0)
    i = pl.program_id(1)

    @pl.when((t == 0) & (i == 0))
    def _init():
        state_ref[...] = state0_ref[...]
        total_ref[...] = jnp.zeros_like(total_ref)
        count_ref[...] = jnp.zeros_like(count_ref)

    @pl.when(t == 0)
    def _first_iter():
        zbf = z_ref[...].astype(jnp.bfloat16)
        res_ref[i] = zbf
        _kmeans_step(zbf, i, state_ref, total_ref, count_ref)

    @pl.when((t > 0) & (t < _ITERS))
    def _resident_iter():
        _kmeans_step(res_ref[i], i, state_ref, total_ref, count_ref)

    @pl.when(t == _ITERS)
    def _final():
        stT = state_ref[...]
        zb = z_ref[...]
        z_sq = jnp.sum(zb * zb, axis=1, keepdims=True)
        sc = jax.lax.dot_general(
            zb, stT, (((1,), (0,)), ((), ())),
            preferred_element_type=jnp.float32)
        ssq = jnp.sum(stT * stT, axis=0, keepdims=True)
        nsq = jnp.maximum(z_sq - 2.0 * sc + ssq, 0.0)
        measure = jnp.sqrt(jnp.min(nsq, axis=1, keepdims=True))
        m2 = measure * measure
        c_base = 2.0 * _D
        beta = jnp.zeros_like(m2)
        for scale in (0.1, 0.2, 0.5, 1.0, 2.0, 5.0, 10.0):
            c = c_base * scale
            beta = beta + c / (c + m2)
        pen_ref[...] = _BAND * jnp.log(jnp.abs(beta) + _EPS)

        u = u_ref[...]
        w = w_ref[...]
        uw = jnp.sum(u * w)
        muw = -1.0 + jax.nn.softplus(uw)
        uhat = u + (muw - uw) * w / jnp.sum(w * w)
        zwb = jnp.sum(zb * w, axis=1, keepdims=True) + b_ref[0, 0]
        th = jnp.tanh(zwb)
        fz_ref[...] = zb + th * uhat
        wu = jnp.sum(w * uhat)
        psi_u = (1.0 - th * th) * wu
        ld_ref[...] = jnp.log(jnp.abs(1.0 + psi_u) + _EPS)


@functools.partial(jax.jit, static_argnames=("interpret",))
def _run(z, u, w, b, interpret=False):
    p = jax.random.uniform(jax.random.key(42), (z.shape[0],),
                           minval=0.0, maxval=1.0)
    _, ind = jax.lax.top_k(p, _N_CLUSTERS)
    state0 = jnp.take(z, ind, axis=0).T

    u2 = u.reshape(1, _D)
    w2 = w.reshape(1, _D)
    b2 = b.reshape(1, 1)

    def _const_map(t, i):
        return (0, 0)

    def _z_map(t, i):
        return (jnp.where((t == 0) | (t == _ITERS), i, 0), 0)

    def _out_map(t, i):
        return (jnp.where(t == _ITERS, i, 0), 0)

    fz, ld, pen = pl.pallas_call(
        _body,
        grid=(_ITERS + 1, _NB),
        in_specs=[
            pl.BlockSpec((_D, _N_CLUSTERS), _const_map),
            pl.BlockSpec((1, _D), _const_map),
            pl.BlockSpec((1, _D), _const_map),
            pl.BlockSpec((1, 1), _const_map),
            pl.BlockSpec((_BR, _D), _z_map),
        ],
        out_specs=[
            pl.BlockSpec((_BR, _D), _out_map),
            pl.BlockSpec((_BR, 1), _out_map),
            pl.BlockSpec((_BR, 1), _out_map),
        ],
        out_shape=[
            jax.ShapeDtypeStruct((_BATCH, _D), jnp.float32),
            jax.ShapeDtypeStruct((_BATCH, 1), jnp.float32),
            jax.ShapeDtypeStruct((_BATCH, 1), jnp.float32),
        ],
        scratch_shapes=[
            pltpu.VMEM((_D, _N_CLUSTERS), jnp.float32),
            pltpu.VMEM((_D, _N_CLUSTERS), jnp.float32),
            pltpu.VMEM((1, _N_CLUSTERS), jnp.float32),
            pltpu.VMEM((_NB, _BR, _D), jnp.bfloat16),
        ],
        compiler_params=pltpu.CompilerParams(
            dimension_semantics=("arbitrary", "arbitrary"),
        ),
        interpret=interpret,
    )(state0, u2, w2, b2, z)
    return fz, ld.reshape(-1), pen.reshape(-1)


def kernel(z, u, w, b):
    return _run(z, u, w, b)


# speedup vs baseline: 1.7246x; 1.7246x over previous
import functools

import jax
import jax.numpy as jnp
from jax.experimental import pallas as pl
from jax.experimental.pallas import tpu as pltpu

_BATCH = 8192
_D = 2048
_N_CLUSTERS = 3
_ITERS = 20
_EPS = 1e-7
_BAND = 0.01
_BR = 512
_NB = _BATCH // _BR


def _kmeans_step(ztb, i, state_ref, total_ref, count_ref):
    st = state_ref[...]
    scT = jax.lax.dot_general(
        st.astype(jnp.bfloat16), ztb, (((1,), (0,)), ((), ())),
        preferred_element_type=jnp.float32)
    ssq = jnp.sum(st * st, axis=1, keepdims=True)
    dsT = ssq - 2.0 * scT
    d0 = dsT[0:1, :]
    d1 = dsT[1:2, :]
    d2 = dsT[2:3, :]
    m0 = (d0 <= d1) & (d0 <= d2)
    m1 = (d1 < d0) & (d1 <= d2)
    m2 = jnp.logical_not(m0 | m1)
    ohT = jnp.concatenate(
        [m0.astype(jnp.float32), m1.astype(jnp.float32),
         m2.astype(jnp.float32)], axis=0)
    count_ref[...] += jnp.sum(ohT, axis=1, keepdims=True)
    total_ref[...] += jax.lax.dot_general(
        ztb, ohT.astype(jnp.bfloat16), (((1,), (1,)), ((), ())),
        preferred_element_type=jnp.float32)

    @pl.when(i == _NB - 1)
    def _update_centers():
        tot = total_ref[...].T
        state_ref[...] = tot / count_ref[:, 0:1]
        total_ref[...] = jnp.zeros_like(total_ref)
        count_ref[...] = jnp.zeros_like(count_ref)


def _body(state0_ref, u_ref, w_ref, b_ref, zt_ref, z_ref,
          fz_ref, ld_ref, pen_ref,
          state_ref, total_ref, count_ref, res_ref):
    t = pl.program_id(0)
    i = pl.program_id(1)

    @pl.when((t == 0) & (i == 0))
    def _init():
        state_ref[...] = state0_ref[...]
        total_ref[...] = jnp.zeros_like(total_ref)
        count_ref[...] = jnp.zeros_like(count_ref)

    @pl.when(t == 0)
    def _first_iter():
        ztb = zt_ref[...]
        res_ref[i] = ztb
        _kmeans_step(ztb, i, state_ref, total_ref, count_ref)

    @pl.when((t > 0) & (t < _ITERS))
    def _resident_iter():
        _kmeans_step(res_ref[i], i, state_ref, total_ref, count_ref)

    @pl.when(t == _ITERS)
    def _final():
        st = state_ref[...]
        zb = z_ref[...]
        z_sq = jnp.sum(zb * zb, axis=1, keepdims=True)
        sc = jax.lax.dot_general(
            zb, st, (((1,), (1,)), ((), ())),
            preferred_element_type=jnp.float32)
        ssq = jnp.sum(st * st, axis=1, keepdims=True)
        nsq = jnp.maximum(z_sq - 2.0 * sc + ssq.T, 0.0)
        measure = jnp.sqrt(jnp.min(nsq, axis=1, keepdims=True))
        m2 = measure * measure
        c_base = 2.0 * _D
        beta = jnp.zeros_like(m2)
        for scale in (0.1, 0.2, 0.5, 1.0, 2.0, 5.0, 10.0):
            c = c_base * scale
            beta = beta + c / (c + m2)
        pen_ref[...] = _BAND * jnp.log(jnp.abs(beta) + _EPS)

        u = u_ref[...]
        w = w_ref[...]
        uw = jnp.sum(u * w)
        muw = -1.0 + jax.nn.softplus(uw)
        uhat = u + (muw - uw) * w / jnp.sum(w * w)
        zwb = jnp.sum(zb * w, axis=1, keepdims=True) + b_ref[0, 0]
        th = jnp.tanh(zwb)
        fz_ref[...] = zb + th * uhat
        wu = jnp.sum(w * uhat)
        psi_u = (1.0 - th * th) * wu
        ld_ref[...] = jnp.log(jnp.abs(1.0 + psi_u) + _EPS)


@functools.partial(jax.jit, static_argnames=("interpret",))
def _run(z, u, w, b, interpret=False):
    p = jax.random.uniform(jax.random.key(42), (z.shape[0],),
                           minval=0.0, maxval=1.0)
    _, ind = jax.lax.top_k(p, _N_CLUSTERS)
    state0 = jnp.take(z, ind, axis=0)
    zt = z.T.astype(jnp.bfloat16)

    u2 = u.reshape(1, _D)
    w2 = w.reshape(1, _D)
    b2 = b.reshape(1, 1)

    def _const_map(t, i):
        return (0, 0)

    def _zt_map(t, i):
        return (0, jnp.where(t == 0, i, 0))

    def _z_map(t, i):
        return (jnp.where(t == _ITERS, i, 0), 0)

    def _out_map(t, i):
        return (jnp.where(t == _ITERS, i, 0), 0)

    fz, ld, pen = pl.pallas_call(
        _body,
        grid=(_ITERS + 1, _NB),
        in_specs=[
            pl.BlockSpec((_N_CLUSTERS, _D), _const_map),
            pl.BlockSpec((1, _D), _const_map),
            pl.BlockSpec((1, _D), _const_map),
            pl.BlockSpec((1, 1), _const_map),
            pl.BlockSpec((_D, _BR), _zt_map),
            pl.BlockSpec((_BR, _D), _z_map),
        ],
        out_specs=[
            pl.BlockSpec((_BR, _D), _out_map),
            pl.BlockSpec((_BR, 1), _out_map),
            pl.BlockSpec((_BR, 1), _out_map),
        ],
        out_shape=[
            jax.ShapeDtypeStruct((_BATCH, _D), jnp.float32),
            jax.ShapeDtypeStruct((_BATCH, 1), jnp.float32),
            jax.ShapeDtypeStruct((_BATCH, 1), jnp.float32),
        ],
        scratch_shapes=[
            pltpu.VMEM((_N_CLUSTERS, _D), jnp.float32),
            pltpu.VMEM((_D, _N_CLUSTERS), jnp.float32),
            pltpu.VMEM((_N_CLUSTERS, 128), jnp.float32),
            pltpu.VMEM((_NB, _D, _BR), jnp.bfloat16),
        ],
        compiler_params=pltpu.CompilerParams(
            dimension_semantics=("arbitrary", "arbitrary"),
        ),
        interpret=interpret,
    )(state0, u2, w2, b2, zt, z)
    return fz, ld.reshape(-1), pen.reshape(-1)


def kernel(z, u, w, b):
    return _run(z, u, w, b)


# speedup vs baseline: 2.0511x; 1.1894x over previous
import functools

import jax
import jax.numpy as jnp
from jax.experimental import pallas as pl
from jax.experimental.pallas import tpu as pltpu

_BATCH = 8192
_D = 2048
_N_CLUSTERS = 3
_ITERS = 20
_EPS = 1e-7
_BAND = 0.01
_BR = 512
_NB = _BATCH // _BR


def _kmeans_step(ztb, i, state_ref, total_ref, count_ref):
    st = state_ref[...]
    scT = jax.lax.dot_general(
        st.astype(jnp.bfloat16), ztb, (((1,), (0,)), ((), ())),
        preferred_element_type=jnp.float32)
    ssq = jnp.sum(st * st, axis=1, keepdims=True)
    dsT = ssq - 2.0 * scT
    d0 = dsT[0:1, :]
    d1 = dsT[1:2, :]
    d2 = dsT[2:3, :]
    m0 = (d0 <= d1) & (d0 <= d2)
    m1 = (d1 < d0) & (d1 <= d2)
    m2 = jnp.logical_not(m0 | m1)
    ohT = jnp.concatenate(
        [m0.astype(jnp.float32), m1.astype(jnp.float32),
         m2.astype(jnp.float32)], axis=0)
    count_ref[...] += jnp.sum(ohT, axis=1, keepdims=True)
    total_ref[...] += jax.lax.dot_general(
        ohT.astype(jnp.bfloat16), ztb, (((1,), (1,)), ((), ())),
        preferred_element_type=jnp.float32)

    @pl.when(i == _NB - 1)
    def _update_centers():
        state_ref[...] = total_ref[...] / count_ref[:, 0:1]
        total_ref[...] = jnp.zeros_like(total_ref)
        count_ref[...] = jnp.zeros_like(count_ref)


def _body(state0_ref, u_ref, w_ref, b_ref, zt_ref, z_ref,
          fz_ref, ld_ref, pen_ref,
          state_ref, total_ref, count_ref, res_ref):
    t = pl.program_id(0)
    i = pl.program_id(1)

    @pl.when((t == 0) & (i == 0))
    def _init():
        state_ref[...] = state0_ref[...]
        total_ref[...] = jnp.zeros_like(total_ref)
        count_ref[...] = jnp.zeros_like(count_ref)

    @pl.when(t == 0)
    def _first_iter():
        ztb = zt_ref[...]
        res_ref[i] = ztb
        _kmeans_step(ztb, i, state_ref, total_ref, count_ref)

    @pl.when((t > 0) & (t < _ITERS))
    def _resident_iter():
        _kmeans_step(res_ref[i], i, state_ref, total_ref, count_ref)

    @pl.when(t == _ITERS)
    def _final():
        st = state_ref[...]
        zb = z_ref[...]
        z_sq = jnp.sum(zb * zb, axis=1, keepdims=True)
        sc = jax.lax.dot_general(
            zb, st, (((1,), (1,)), ((), ())),
            preferred_element_type=jnp.float32)
        ssq = jnp.sum(st * st, axis=1, keepdims=True)
        nsq = jnp.maximum(z_sq - 2.0 * sc + ssq.T, 0.0)
        measure = jnp.sqrt(jnp.min(nsq, axis=1, keepdims=True))
        m2 = measure * measure
        c_base = 2.0 * _D
        beta = jnp.zeros_like(m2)
        for scale in (0.1, 0.2, 0.5, 1.0, 2.0, 5.0, 10.0):
            c = c_base * scale
            beta = beta + c / (c + m2)
        pen_ref[...] = _BAND * jnp.log(jnp.abs(beta) + _EPS)

        u = u_ref[...]
        w = w_ref[...]
        uw = jnp.sum(u * w)
        muw = -1.0 + jax.nn.softplus(uw)
        uhat = u + (muw - uw) * w / jnp.sum(w * w)
        zwb = jnp.sum(zb * w, axis=1, keepdims=True) + b_ref[0, 0]
        th = jnp.tanh(zwb)
        fz_ref[...] = zb + th * uhat
        wu = jnp.sum(w * uhat)
        psi_u = (1.0 - th * th) * wu
        ld_ref[...] = jnp.log(jnp.abs(1.0 + psi_u) + _EPS)


@functools.partial(jax.jit, static_argnames=("interpret",))
def _run(z, u, w, b, interpret=False):
    p = jax.random.uniform(jax.random.key(42), (z.shape[0],),
                           minval=0.0, maxval=1.0)
    _, ind = jax.lax.top_k(p, _N_CLUSTERS)
    state0 = jnp.take(z, ind, axis=0)
    zt = z.T.astype(jnp.bfloat16)

    u2 = u.reshape(1, _D)
    w2 = w.reshape(1, _D)
    b2 = b.reshape(1, 1)

    def _const_map(t, i):
        return (0, 0)

    def _zt_map(t, i):
        return (0, jnp.where(t == 0, i, 0))

    def _z_map(t, i):
        return (jnp.where(t == _ITERS, i, 0), 0)

    def _out_map(t, i):
        return (jnp.where(t == _ITERS, i, 0), 0)

    fz, ld, pen = pl.pallas_call(
        _body,
        grid=(_ITERS + 1, _NB),
        in_specs=[
            pl.BlockSpec((_N_CLUSTERS, _D), _const_map),
            pl.BlockSpec((1, _D), _const_map),
            pl.BlockSpec((1, _D), _const_map),
            pl.BlockSpec((1, 1), _const_map),
            pl.BlockSpec((_D, _BR), _zt_map),
            pl.BlockSpec((_BR, _D), _z_map),
        ],
        out_specs=[
            pl.BlockSpec((_BR, _D), _out_map),
            pl.BlockSpec((_BR, 1), _out_map),
            pl.BlockSpec((_BR, 1), _out_map),
        ],
        out_shape=[
            jax.ShapeDtypeStruct((_BATCH, _D), jnp.float32),
            jax.ShapeDtypeStruct((_BATCH, 1), jnp.float32),
            jax.ShapeDtypeStruct((_BATCH, 1), jnp.float32),
        ],
        scratch_shapes=[
            pltpu.VMEM((_N_CLUSTERS, _D), jnp.float32),
            pltpu.VMEM((_N_CLUSTERS, _D), jnp.float32),
            pltpu.VMEM((_N_CLUSTERS, 128), jnp.float32),
            pltpu.VMEM((_NB, _D, _BR), jnp.bfloat16),
        ],
        compiler_params=pltpu.CompilerParams(
            dimension_semantics=("arbitrary", "arbitrary"),
        ),
        interpret=interpret,
    )(state0, u2, w2, b2, zt, z)
    return fz, ld.reshape(-1), pen.reshape(-1)


def kernel(z, u, w, b):
    return _run(z, u, w, b)
